# Initial kernel scaffold; baseline (speedup 1.0000x reference)
import jax
import jax.numpy as jnp
from jax.experimental import pallas as pl


def kernel(hidden_states, W):
    raise NotImplementedError("write your pallas kernel here")


# speedup vs baseline: 2.1061x; 2.1061x over previous
import jax
import jax.numpy as jnp
from jax import lax
from jax.experimental import pallas as pl
from jax.experimental.pallas import tpu as pltpu

_EXPERTS = 64
_TOPK = 2


def _router_body(h_ref, wt_ref, gates_ref, idx_ref, loss_ref, acc_sum, acc_pos):
    pid = pl.program_id(0)
    nprog = pl.num_programs(0)

    @pl.when(pid == 0)
    def _init():
        acc_sum[...] = jnp.zeros_like(acc_sum)
        acc_pos[...] = jnp.zeros_like(acc_pos)

    logits = jnp.dot(h_ref[...], wt_ref[...],
                     preferred_element_type=jnp.float32)
    tm, e = logits.shape
    iota = lax.broadcasted_iota(jnp.int32, (tm, e), 1)

    m1 = jnp.max(logits, axis=1, keepdims=True)
    i1 = jnp.min(jnp.where(logits == m1, iota, e), axis=1, keepdims=True)
    masked = jnp.where(iota == i1, jnp.float32(-jnp.inf), logits)
    m2 = jnp.max(masked, axis=1, keepdims=True)
    i2 = jnp.min(jnp.where(masked == m2, iota, e), axis=1, keepdims=True)

    e2 = jnp.exp(m2 - m1)
    denom = 1.0 + e2
    gates_ref[...] = jnp.concatenate([1.0 / denom, e2 / denom], axis=1)
    idx_ref[...] = jnp.concatenate([i1, i2], axis=1)

    p = jnp.exp(logits - m1)
    pn = p / jnp.sum(p, axis=1, keepdims=True)
    acc_sum[...] += jnp.sum(pn, axis=0, keepdims=True)
    acc_pos[...] += jnp.sum((pn > 0).astype(jnp.float32), axis=0, keepdims=True)

    @pl.when(pid == nprog - 1)
    def _fin():
        n_tok = jnp.float32(nprog * tm)
        loss = (jnp.float32(e) / (n_tok * n_tok)) * jnp.sum(
            acc_sum[...] * acc_pos[...], keepdims=True)
        loss_ref[...] = loss.reshape(1, 1)


def _run(hidden_flat, wt, tm, interpret=False):
    n, h = hidden_flat.shape
    e = wt.shape[1]
    grid = (n // tm,)
    return pl.pallas_call(
        _router_body,
        grid=grid,
        in_specs=[
            pl.BlockSpec((tm, h), lambda i: (i, 0)),
            pl.BlockSpec((h, e), lambda i: (0, 0)),
        ],
        out_specs=[
            pl.BlockSpec((tm, _TOPK), lambda i: (i, 0)),
            pl.BlockSpec((tm, _TOPK), lambda i: (i, 0)),
            pl.BlockSpec((1, 1), lambda i: (0, 0)),
        ],
        out_shape=[
            jax.ShapeDtypeStruct((n, _TOPK), jnp.float32),
            jax.ShapeDtypeStruct((n, _TOPK), jnp.int32),
            jax.ShapeDtypeStruct((1, 1), jnp.float32),
        ],
        scratch_shapes=[
            pltpu.VMEM((1, e), jnp.float32),
            pltpu.VMEM((1, e), jnp.float32),
        ],
        compiler_params=pltpu.CompilerParams(
            dimension_semantics=("arbitrary",)),
        interpret=interpret,
    )(hidden_flat, wt)


def kernel(hidden_states, W):
    b, s, h = hidden_states.shape
    hf = hidden_states.reshape(b * s, h)
    gates, idx, loss = _run(hf, W.T, tm=1024)
    return (gates.reshape(b, s, _TOPK), idx.reshape(b, s, _TOPK), loss[0, 0])


# speedup vs baseline: 2.3628x; 1.1219x over previous
import jax
import jax.numpy as jnp
from jax import lax
from jax.experimental import pallas as pl
from jax.experimental.pallas import tpu as pltpu

_EXPERTS = 64
_TOPK = 2


def _router_body(h_ref, wt_ref, gates_ref, idx_ref, loss_ref, acc_sum, acc_pos):
    pid = pl.program_id(0)
    nprog = pl.num_programs(0)

    @pl.when(pid == 0)
    def _init():
        acc_sum[...] = jnp.zeros_like(acc_sum)
        acc_pos[...] = jnp.zeros_like(acc_pos)

    logits = jnp.dot(h_ref[...], wt_ref[...],
                     preferred_element_type=jnp.float32)
    tm, e = logits.shape
    iota = lax.broadcasted_iota(jnp.int32, (tm, e), 1)

    m1 = jnp.max(logits, axis=1, keepdims=True)
    i1 = jnp.min(jnp.where(logits == m1, iota, e), axis=1, keepdims=True)
    masked = jnp.where(iota == i1, jnp.float32(-jnp.inf), logits)
    m2 = jnp.max(masked, axis=1, keepdims=True)
    i2 = jnp.min(jnp.where(masked == m2, iota, e), axis=1, keepdims=True)

    e2 = jnp.exp(m2 - m1)
    denom = 1.0 + e2
    gates_ref[...] = jnp.concatenate([1.0 / denom, e2 / denom], axis=1)
    idx_ref[...] = jnp.concatenate([i1, i2], axis=1)

    p = jnp.exp(logits - m1)
    pn = p / jnp.sum(p, axis=1, keepdims=True)
    acc_sum[...] += jnp.sum(pn, axis=0, keepdims=True)
    acc_pos[...] += jnp.sum((pn > 0).astype(jnp.float32), axis=0, keepdims=True)

    @pl.when(pid == nprog - 1)
    def _fin():
        n_tok = jnp.float32(nprog * tm)
        loss = (jnp.float32(e) / (n_tok * n_tok)) * jnp.sum(
            acc_sum[...] * acc_pos[...], keepdims=True)
        loss_ref[...] = loss.reshape(1, 1)


def _run(hidden_flat, wt, tm, interpret=False):
    n, h = hidden_flat.shape
    e = wt.shape[1]
    grid = (n // tm,)
    return pl.pallas_call(
        _router_body,
        grid=grid,
        in_specs=[
            pl.BlockSpec((tm, h), lambda i: (i, 0)),
            pl.BlockSpec((h, e), lambda i: (0, 0)),
        ],
        out_specs=[
            pl.BlockSpec((tm, _TOPK), lambda i: (i, 0)),
            pl.BlockSpec((tm, _TOPK), lambda i: (i, 0)),
            pl.BlockSpec((1, 1), lambda i: (0, 0)),
        ],
        out_shape=[
            jax.ShapeDtypeStruct((n, _TOPK), jnp.float32),
            jax.ShapeDtypeStruct((n, _TOPK), jnp.int32),
            jax.ShapeDtypeStruct((1, 1), jnp.float32),
        ],
        scratch_shapes=[
            pltpu.VMEM((1, e), jnp.float32),
            pltpu.VMEM((1, e), jnp.float32),
        ],
        compiler_params=pltpu.CompilerParams(
            dimension_semantics=("arbitrary",)),
        interpret=interpret,
    )(hidden_flat, wt)


def kernel(hidden_states, W):
    b, s, h = hidden_states.shape
    hf = hidden_states.reshape(b * s, h)
    gates, idx, loss = _run(hf, W.T, tm=2048)
    return (gates.reshape(b, s, _TOPK), idx.reshape(b, s, _TOPK), loss[0, 0])


# speedup vs baseline: 2.5372x; 1.0738x over previous
import jax
import jax.numpy as jnp
from jax import lax
from jax.experimental import pallas as pl
from jax.experimental.pallas import tpu as pltpu

_EXPERTS = 64
_TOPK = 2


def _router_body(h_ref, wt_ref, gates_ref, idx_ref, loss_ref, acc_sum, acc_pos):
    pid = pl.program_id(0)
    nprog = pl.num_programs(0)

    @pl.when(pid == 0)
    def _init():
        acc_sum[...] = jnp.zeros_like(acc_sum)
        acc_pos[...] = jnp.zeros_like(acc_pos)

    logits = jnp.dot(h_ref[...], wt_ref[...],
                     preferred_element_type=jnp.float32)
    tm, e = logits.shape
    iota = lax.broadcasted_iota(jnp.int32, (tm, e), 1)

    m1 = jnp.max(logits, axis=1, keepdims=True)
    i1 = jnp.min(jnp.where(logits == m1, iota, e), axis=1, keepdims=True)
    masked = jnp.where(iota == i1, jnp.float32(-jnp.inf), logits)
    m2 = jnp.max(masked, axis=1, keepdims=True)
    i2 = jnp.min(jnp.where(masked == m2, iota, e), axis=1, keepdims=True)

    e2 = jnp.exp(m2 - m1)
    denom = 1.0 + e2
    gates_ref[...] = jnp.concatenate([1.0 / denom, e2 / denom], axis=1)
    idx_ref[...] = jnp.concatenate([i1, i2], axis=1)

    p = jnp.exp(logits - m1)
    pn = p / jnp.sum(p, axis=1, keepdims=True)
    acc_sum[...] += jnp.sum(pn, axis=0, keepdims=True)
    acc_pos[...] += jnp.sum((pn > 0).astype(jnp.float32), axis=0, keepdims=True)

    @pl.when(pid == nprog - 1)
    def _fin():
        n_tok = jnp.float32(nprog * tm)
        loss = (jnp.float32(e) / (n_tok * n_tok)) * jnp.sum(
            acc_sum[...] * acc_pos[...], keepdims=True)
        loss_ref[...] = loss.reshape(1, 1)


def _run(hidden_flat, wt, tm, interpret=False):
    n, h = hidden_flat.shape
    e = wt.shape[1]
    grid = (n // tm,)
    return pl.pallas_call(
        _router_body,
        grid=grid,
        in_specs=[
            pl.BlockSpec((tm, h), lambda i: (i, 0)),
            pl.BlockSpec((h, e), lambda i: (0, 0)),
        ],
        out_specs=[
            pl.BlockSpec((tm, _TOPK), lambda i: (i, 0)),
            pl.BlockSpec((tm, _TOPK), lambda i: (i, 0)),
            pl.BlockSpec((1, 1), lambda i: (0, 0)),
        ],
        out_shape=[
            jax.ShapeDtypeStruct((n, _TOPK), jnp.float32),
            jax.ShapeDtypeStruct((n, _TOPK), jnp.int32),
            jax.ShapeDtypeStruct((1, 1), jnp.float32),
        ],
        scratch_shapes=[
            pltpu.VMEM((1, e), jnp.float32),
            pltpu.VMEM((1, e), jnp.float32),
        ],
        compiler_params=pltpu.CompilerParams(
            dimension_semantics=("arbitrary",)),
        interpret=interpret,
    )(hidden_flat, wt)


def kernel(hidden_states, W):
    b, s, h = hidden_states.shape
    hf = hidden_states.reshape(b * s, h)
    gates, idx, loss = _run(hf, W.T, tm=4096)
    return (gates.reshape(b, s, _TOPK), idx.reshape(b, s, _TOPK), loss[0, 0])


# speedup vs baseline: 5.0878x; 2.0053x over previous
import jax
import jax.numpy as jnp
from jax import lax
from jax.experimental import pallas as pl
from jax.experimental.pallas import tpu as pltpu

_EXPERTS = 64
_TOPK = 2


def _router_body(w_ref, h_ref, gates_ref, idx_ref, loss_ref, acc_sum, acc_pos):
    pid = pl.program_id(0)
    nprog = pl.num_programs(0)

    @pl.when(pid == 0)
    def _init():
        acc_sum[...] = jnp.zeros_like(acc_sum)
        acc_pos[...] = jnp.zeros_like(acc_pos)

    logits = lax.dot_general(
        w_ref[...], h_ref[...],
        dimension_numbers=(((1,), (1,)), ((), ())),
        preferred_element_type=jnp.float32)
    e, tm = logits.shape
    row = lax.broadcasted_iota(jnp.int32, (e, tm), 0)

    m1 = jnp.max(logits, axis=0, keepdims=True)
    i1 = jnp.min(jnp.where(logits == m1, row, e), axis=0, keepdims=True)
    masked = jnp.where(row == i1, jnp.float32(-jnp.inf), logits)
    m2 = jnp.max(masked, axis=0, keepdims=True)
    i2 = jnp.min(jnp.where(masked == m2, row, e), axis=0, keepdims=True)

    e2 = jnp.exp(m2 - m1)
    denom = 1.0 + e2
    gates_ref[...] = jnp.concatenate([1.0 / denom, e2 / denom], axis=0)
    idx_ref[...] = jnp.concatenate([i1, i2], axis=0)

    p = jnp.exp(logits - m1)
    pn = p / jnp.sum(p, axis=0, keepdims=True)
    acc_sum[...] += pn
    acc_pos[...] += (pn > 0).astype(jnp.float32)

    @pl.when(pid == nprog - 1)
    def _fin():
        n_tok = jnp.float32(nprog * tm)
        s_e = jnp.sum(acc_sum[...], axis=1)
        c_e = jnp.sum(acc_pos[...], axis=1)
        loss = (jnp.float32(e) / (n_tok * n_tok)) * jnp.sum(
            s_e * c_e, keepdims=True)
        loss_ref[...] = loss.reshape(1, 1)


def _run(hidden_flat, w, tm, interpret=False):
    n, h = hidden_flat.shape
    e = w.shape[0]
    grid = (n // tm,)
    return pl.pallas_call(
        _router_body,
        grid=grid,
        in_specs=[
            pl.BlockSpec((e, h), lambda i: (0, 0)),
            pl.BlockSpec((tm, h), lambda i: (i, 0)),
        ],
        out_specs=[
            pl.BlockSpec((_TOPK, tm), lambda i: (0, i)),
            pl.BlockSpec((_TOPK, tm), lambda i: (0, i)),
            pl.BlockSpec((1, 1), lambda i: (0, 0)),
        ],
        out_shape=[
            jax.ShapeDtypeStruct((_TOPK, n), jnp.float32),
            jax.ShapeDtypeStruct((_TOPK, n), jnp.int32),
            jax.ShapeDtypeStruct((1, 1), jnp.float32),
        ],
        scratch_shapes=[
            pltpu.VMEM((e, tm), jnp.float32),
            pltpu.VMEM((e, tm), jnp.float32),
        ],
        compiler_params=pltpu.CompilerParams(
            dimension_semantics=("arbitrary",)),
        interpret=interpret,
    )(w, hidden_flat)


def kernel(hidden_states, W):
    b, s, h = hidden_states.shape
    hf = hidden_states.reshape(b * s, h)
    gates_t, idx_t, loss = _run(hf, W, tm=4096)
    gates = gates_t.T.reshape(b, s, _TOPK)
    idx = idx_t.T.reshape(b, s, _TOPK)
    return (gates, idx, loss[0, 0])
